# baseline (speedup 1.0000x reference)
import functools

import jax
import jax.numpy as jnp
from jax import lax
from jax.experimental import pallas as pl
from jax.experimental.pallas import tpu as pltpu
from jax.experimental.pallas import tpu_sc as plsc

N = 10000
E = 320000
D_IN = 128
DIM = 256
G = 128

NC = 2
NS = 16
DH = D_IN // NC
CHUNK = 128
CHUNKS_PER_S = 160
GROUP = 2
GROUPS = CHUNKS_PER_S // GROUP
E_PAD = NS * CHUNKS_PER_S * CHUNK
N_PAD = N + 8
ROWS_PER_S = 624
TAIL0 = NS * ROWS_PER_S
TAIL = N - TAIL0

BLK = 400
NB = N // BLK
BN_INV = 1.0 / (1.0 + 1e-5) ** 0.5


def _sc_agg_body(x_hbm, src_hbm, dst_hbm, out_hbm,
                 src_v, dst_v, rows, acc, gsA, gsB, ssem):
    c = lax.axis_index("c")
    s = lax.axis_index("s")
    pltpu.sync_copy(src_hbm.at[c * NS + s], src_v)
    pltpu.sync_copy(dst_hbm.at[s], dst_v)
    row0 = s * ROWS_PER_S
    pltpu.sync_copy(x_hbm.at[pl.ds(c * N + row0, ROWS_PER_S)],
                    acc.at[pl.ds(row0, ROWS_PER_S)])

    @pl.when(s == 0)
    def _():
        pltpu.sync_copy(x_hbm.at[pl.ds(c * N + TAIL0, TAIL)],
                        acc.at[pl.ds(TAIL0, TAIL)])

    plsc.subcore_barrier()

    def fire_gathers(g, bank, gsem):
        for k in range(GROUP):
            pltpu.async_copy(x_hbm.at[src_v.at[GROUP * g + k]],
                             rows.at[bank * GROUP + k], gsem)

    def drain_gathers(gsem):
        for k in range(GROUP):
            pltpu.make_async_copy(x_hbm.at[src_v.at[k]], rows.at[k], gsem).wait()

    def fire_scatters(g, bank, ssem):
        for k in range(GROUP):
            pltpu.async_copy(rows.at[bank * GROUP + k],
                             acc.at[dst_v.at[GROUP * g + k]], ssem, add=True)

    def drain_scatters(ssem):
        for k in range(GROUP):
            pltpu.make_async_copy(rows.at[k], acc.at[dst_v.at[k]], ssem).wait()

    fire_gathers(0, 0, gsA)

    def pair(t, carry):
        gA = 2 * t
        gB = gA + 1

        @pl.when(t > 0)
        def _():
            drain_scatters(ssem)

        fire_gathers(gB, 1, gsB)
        drain_gathers(gsA)
        fire_scatters(gA, 0, ssem)
        drain_scatters(ssem)

        @pl.when(t < GROUPS // 2 - 1)
        def _():
            fire_gathers(gA + 2, 0, gsA)

        drain_gathers(gsB)
        fire_scatters(gB, 1, ssem)
        return carry

    lax.fori_loop(0, GROUPS // 2, pair, 0)
    drain_scatters(ssem)
    plsc.subcore_barrier()
    pltpu.sync_copy(acc.at[pl.ds(row0, ROWS_PER_S)],
                    out_hbm.at[c, pl.ds(row0, ROWS_PER_S)])

    @pl.when(s == 0)
    def _():
        pltpu.sync_copy(acc.at[pl.ds(TAIL0, TAIL)],
                        out_hbm.at[c, pl.ds(TAIL0, TAIL)])


@functools.lru_cache(maxsize=1)
def _sc_agg():
    mesh = plsc.VectorSubcoreMesh(core_axis_name="c", subcore_axis_name="s")
    return pl.kernel(
        _sc_agg_body,
        out_type=jax.ShapeDtypeStruct((NC, N, DH), jnp.float32),
        mesh=mesh,
        scratch_types=[
            pltpu.VMEM((CHUNKS_PER_S, CHUNK), jnp.int32),
            pltpu.VMEM((CHUNKS_PER_S, CHUNK), jnp.int32),
            pltpu.VMEM((2 * GROUP, CHUNK, DH), jnp.float32),
            pltpu.VMEM_SHARED((N_PAD, DH), jnp.float32),
            pltpu.SemaphoreType.DMA,
            pltpu.SemaphoreType.DMA,
            pltpu.SemaphoreType.DMA,
        ],
        compiler_params=pltpu.CompilerParams(use_tc_tiling_on_sc=False),
    )


def _tc_head_body(a_ref, b_ref, batch_ref,
                  w1_ref, b1_ref, w2_ref, b2_ref, bng_ref, bnb_ref,
                  fc1w_ref, fc1b_ref, fc2w_ref, fc2b_ref,
                  out_ref, acc_ref):
    i = pl.program_id(0)

    @pl.when(i == 0)
    def _():
        acc_ref[...] = jnp.zeros_like(acc_ref)

    h = jnp.concatenate([a_ref[...], b_ref[...]], axis=1)
    h = jnp.maximum(
        jax.lax.dot(h, w1_ref[...], preferred_element_type=jnp.float32)
        + b1_ref[...], 0.0)
    h = jax.lax.dot(h, w2_ref[...], preferred_element_type=jnp.float32) \
        + b2_ref[...]
    h = jnp.maximum(h, 0.0)
    h = h * (BN_INV * bng_ref[...]) + bnb_ref[...]

    bvec = batch_ref[0, 0, :]
    onehot = jnp.where(
        bvec[:, None] == jax.lax.broadcasted_iota(jnp.int32, (BLK, G), 1),
        1.0, 0.0)
    acc_ref[...] += jax.lax.dot_general(
        onehot, h, (((0,), (0,)), ((), ())),
        preferred_element_type=jnp.float32)

    @pl.when(i == NB - 1)
    def _():
        g = jnp.maximum(
            jax.lax.dot(acc_ref[...], fc1w_ref[...],
                        preferred_element_type=jnp.float32) + fc1b_ref[...],
            0.0)
        out_ref[...] = jax.lax.dot(
            g, fc2w_ref[...], preferred_element_type=jnp.float32) + fc2b_ref[...]


_tc_head = pl.pallas_call(
    _tc_head_body,
    grid=(NB,),
    in_specs=[
        pl.BlockSpec((BLK, DH), lambda i: (i, 0)),
        pl.BlockSpec((BLK, DH), lambda i: (i, 0)),
        pl.BlockSpec((1, 1, BLK), lambda i: (i, 0, 0)),
        pl.BlockSpec((D_IN, DIM), lambda i: (0, 0)),
        pl.BlockSpec((1, DIM), lambda i: (0, 0)),
        pl.BlockSpec((DIM, DIM), lambda i: (0, 0)),
        pl.BlockSpec((1, DIM), lambda i: (0, 0)),
        pl.BlockSpec((1, DIM), lambda i: (0, 0)),
        pl.BlockSpec((1, DIM), lambda i: (0, 0)),
        pl.BlockSpec((DIM, DIM), lambda i: (0, 0)),
        pl.BlockSpec((1, DIM), lambda i: (0, 0)),
        pl.BlockSpec((DIM, G), lambda i: (0, 0)),
        pl.BlockSpec((1, G), lambda i: (0, 0)),
    ],
    out_specs=pl.BlockSpec((G, G), lambda i: (0, 0)),
    out_shape=jax.ShapeDtypeStruct((G, G), jnp.float32),
    scratch_shapes=[pltpu.VMEM((G, DIM), jnp.float32)],
    compiler_params=pltpu.CompilerParams(
        dimension_semantics=("arbitrary",)),
)


def kernel(x, edge_index, batch, W1, b1, W2, b2, bn_g, bn_b,
           fc1_W, fc1_b, fc2_W, fc2_b):
    src = edge_index[0]
    dst = edge_index[1]
    pad = E_PAD - E
    src_p = jnp.concatenate([src, jnp.zeros((pad,), jnp.int32)])
    src_p = src_p.reshape(NS, CHUNKS_PER_S, CHUNK)
    src_p = jnp.concatenate([src_p, src_p + N])
    dst_p = jnp.concatenate(
        [dst, jnp.full((pad,), N, jnp.int32)]).reshape(NS, CHUNKS_PER_S, CHUNK)
    x2 = x.reshape(N, NC, DH).transpose(1, 0, 2).reshape(NC * N, DH)

    hio = _sc_agg()(x2, src_p, dst_p)

    batch3 = batch.reshape(NB, 1, BLK)

    fc2p = jnp.pad(fc2_W, ((0, 0), (0, G - 1)))
    fc2bp = jnp.pad(fc2_b, (0, G - 1)).reshape(1, G)

    out = _tc_head(hio[0], hio[1], batch3,
                   W1, b1.reshape(1, DIM), W2, b2.reshape(1, DIM),
                   bn_g.reshape(1, DIM), bn_b.reshape(1, DIM),
                   fc1_W, fc1_b.reshape(1, DIM), fc2p, fc2bp)
    return out[:, :1]


# speedup vs baseline: 1.0144x; 1.0144x over previous
import functools

import jax
import jax.numpy as jnp
from jax import lax
from jax.experimental import pallas as pl
from jax.experimental.pallas import tpu as pltpu
from jax.experimental.pallas import tpu_sc as plsc

N = 10000
E = 320000
D_IN = 128
DIM = 256
G = 128

NC = 2
NS = 16
DH = D_IN // NC
CHUNK = 128
CHUNKS_PER_S = 160
E_PAD = NS * CHUNKS_PER_S * CHUNK
N_PAD = N + 8
ROWS_PER_S = 624
TAIL0 = NS * ROWS_PER_S
TAIL = N - TAIL0

BLK = 400
NB = N // BLK
BN_INV = 1.0 / (1.0 + 1e-5) ** 0.5


def _sc_agg_body(x_hbm, src_hbm, dst_hbm, out_hbm,
                 src_v, dst_v, rows, acc, *gsems):
    c = lax.axis_index("c")
    s = lax.axis_index("s")
    pltpu.sync_copy(src_hbm.at[c * NS + s], src_v)
    pltpu.sync_copy(dst_hbm.at[s], dst_v)
    row0 = s * ROWS_PER_S
    pltpu.sync_copy(x_hbm.at[pl.ds(c * N + row0, ROWS_PER_S)],
                    acc.at[pl.ds(row0, ROWS_PER_S)])

    @pl.when(s == 0)
    def _():
        pltpu.sync_copy(x_hbm.at[pl.ds(c * N + TAIL0, TAIL)],
                        acc.at[pl.ds(TAIL0, TAIL)])

    plsc.subcore_barrier()

    def fire(j, k, gsem):
        pltpu.async_copy(x_hbm.at[src_v.at[j]], rows.at[k], gsem)

    def wait(k, gsem):
        pltpu.make_async_copy(x_hbm.at[src_v.at[k]], rows.at[k], gsem).wait()

    for k in range(4):
        fire(k, k, gsems[k])

    def quad(i, carry):
        j0 = 4 * i
        for k in range(4):
            wait(k, gsems[k])
            pltpu.sync_copy(rows.at[k], acc.at[dst_v.at[j0 + k]], add=True)

            @pl.when(j0 + k + 4 < CHUNKS_PER_S)
            def _():
                fire(j0 + k + 4, k, gsems[k])
        return carry

    lax.fori_loop(0, CHUNKS_PER_S // 4, quad, 0)
    plsc.subcore_barrier()
    pltpu.sync_copy(acc.at[pl.ds(row0, ROWS_PER_S)],
                    out_hbm.at[c, pl.ds(row0, ROWS_PER_S)])

    @pl.when(s == 0)
    def _():
        pltpu.sync_copy(acc.at[pl.ds(TAIL0, TAIL)],
                        out_hbm.at[c, pl.ds(TAIL0, TAIL)])


@functools.lru_cache(maxsize=1)
def _sc_agg():
    mesh = plsc.VectorSubcoreMesh(core_axis_name="c", subcore_axis_name="s")
    return pl.kernel(
        _sc_agg_body,
        out_type=jax.ShapeDtypeStruct((NC, N, DH), jnp.float32),
        mesh=mesh,
        scratch_types=[
            pltpu.VMEM((CHUNKS_PER_S, CHUNK), jnp.int32),
            pltpu.VMEM((CHUNKS_PER_S, CHUNK), jnp.int32),
            pltpu.VMEM((4, CHUNK, DH), jnp.float32),
            pltpu.VMEM_SHARED((N_PAD, DH), jnp.float32),
            pltpu.SemaphoreType.DMA,
            pltpu.SemaphoreType.DMA,
            pltpu.SemaphoreType.DMA,
            pltpu.SemaphoreType.DMA,
        ],
        compiler_params=pltpu.CompilerParams(use_tc_tiling_on_sc=False),
    )


def _tc_head_body(a_ref, b_ref, batch_ref,
                  w1_ref, b1_ref, w2_ref, b2_ref, bng_ref, bnb_ref,
                  fc1w_ref, fc1b_ref, fc2w_ref, fc2b_ref,
                  out_ref, acc_ref):
    i = pl.program_id(0)

    @pl.when(i == 0)
    def _():
        acc_ref[...] = jnp.zeros_like(acc_ref)

    h = jnp.concatenate([a_ref[...], b_ref[...]], axis=1)
    h = jnp.maximum(
        jax.lax.dot(h, w1_ref[...], preferred_element_type=jnp.float32)
        + b1_ref[...], 0.0)
    h = jax.lax.dot(h, w2_ref[...], preferred_element_type=jnp.float32) \
        + b2_ref[...]
    h = jnp.maximum(h, 0.0)
    h = h * (BN_INV * bng_ref[...]) + bnb_ref[...]

    bvec = batch_ref[0, 0, :]
    onehot = jnp.where(
        bvec[:, None] == jax.lax.broadcasted_iota(jnp.int32, (BLK, G), 1),
        1.0, 0.0)
    acc_ref[...] += jax.lax.dot_general(
        onehot, h, (((0,), (0,)), ((), ())),
        preferred_element_type=jnp.float32)

    @pl.when(i == NB - 1)
    def _():
        g = jnp.maximum(
            jax.lax.dot(acc_ref[...], fc1w_ref[...],
                        preferred_element_type=jnp.float32) + fc1b_ref[...],
            0.0)
        out_ref[...] = jax.lax.dot(
            g, fc2w_ref[...], preferred_element_type=jnp.float32) + fc2b_ref[...]


_tc_head = pl.pallas_call(
    _tc_head_body,
    grid=(NB,),
    in_specs=[
        pl.BlockSpec((BLK, DH), lambda i: (i, 0)),
        pl.BlockSpec((BLK, DH), lambda i: (i, 0)),
        pl.BlockSpec((1, 1, BLK), lambda i: (i, 0, 0)),
        pl.BlockSpec((D_IN, DIM), lambda i: (0, 0)),
        pl.BlockSpec((1, DIM), lambda i: (0, 0)),
        pl.BlockSpec((DIM, DIM), lambda i: (0, 0)),
        pl.BlockSpec((1, DIM), lambda i: (0, 0)),
        pl.BlockSpec((1, DIM), lambda i: (0, 0)),
        pl.BlockSpec((1, DIM), lambda i: (0, 0)),
        pl.BlockSpec((DIM, DIM), lambda i: (0, 0)),
        pl.BlockSpec((1, DIM), lambda i: (0, 0)),
        pl.BlockSpec((DIM, G), lambda i: (0, 0)),
        pl.BlockSpec((1, G), lambda i: (0, 0)),
    ],
    out_specs=pl.BlockSpec((G, G), lambda i: (0, 0)),
    out_shape=jax.ShapeDtypeStruct((G, G), jnp.float32),
    scratch_shapes=[pltpu.VMEM((G, DIM), jnp.float32)],
    compiler_params=pltpu.CompilerParams(
        dimension_semantics=("arbitrary",)),
)


def kernel(x, edge_index, batch, W1, b1, W2, b2, bn_g, bn_b,
           fc1_W, fc1_b, fc2_W, fc2_b):
    src = edge_index[0]
    dst = edge_index[1]
    pad = E_PAD - E
    src_p = jnp.concatenate([src, jnp.zeros((pad,), jnp.int32)])
    src_p = src_p.reshape(NS, CHUNKS_PER_S, CHUNK)
    src_p = jnp.concatenate([src_p, src_p + N])
    dst_p = jnp.concatenate(
        [dst, jnp.full((pad,), N, jnp.int32)]).reshape(NS, CHUNKS_PER_S, CHUNK)
    x2 = x.reshape(N, NC, DH).transpose(1, 0, 2).reshape(NC * N, DH)

    hio = _sc_agg()(x2, src_p, dst_p)

    batch3 = batch.reshape(NB, 1, BLK)

    fc2p = jnp.pad(fc2_W, ((0, 0), (0, G - 1)))
    fc2bp = jnp.pad(fc2_b, (0, G - 1)).reshape(1, G)

    out = _tc_head(hio[0], hio[1], batch3,
                   W1, b1.reshape(1, DIM), W2, b2.reshape(1, DIM),
                   bn_g.reshape(1, DIM), bn_b.reshape(1, DIM),
                   fc1_W, fc1_b.reshape(1, DIM), fc2p, fc2bp)
    return out[:, :1]


# speedup vs baseline: 1.0148x; 1.0003x over previous
import functools

import jax
import jax.numpy as jnp
from jax import lax
from jax.experimental import pallas as pl
from jax.experimental.pallas import tpu as pltpu
from jax.experimental.pallas import tpu_sc as plsc

N = 10000
E = 320000
D_IN = 128
DIM = 256
G = 128

NC = 2
NS = 16
DH = D_IN // NC
CHUNK = 128
CHUNKS_PER_S = 160
E_PAD = NS * CHUNKS_PER_S * CHUNK
N_PAD = N + 8
ROWS_PER_S = 624
TAIL0 = NS * ROWS_PER_S
TAIL = N - TAIL0

BLK = 400
NB = N // BLK
BN_INV = 1.0 / (1.0 + 1e-5) ** 0.5


def _sc_agg_body(x_hbm, src_hbm, dst_hbm, out_hbm,
                 src_v, dst_v, r0, r1, r2, r3, acc, *gsems):
    c = lax.axis_index("c")
    s = lax.axis_index("s")
    pltpu.sync_copy(src_hbm.at[c * NS + s], src_v)
    pltpu.sync_copy(dst_hbm.at[s], dst_v)
    row0 = s * ROWS_PER_S
    pltpu.sync_copy(x_hbm.at[pl.ds(c * N + row0, ROWS_PER_S)],
                    acc.at[pl.ds(row0, ROWS_PER_S)])

    @pl.when(s == 0)
    def _():
        pltpu.sync_copy(x_hbm.at[pl.ds(c * N + TAIL0, TAIL)],
                        acc.at[pl.ds(TAIL0, TAIL)])

    plsc.subcore_barrier()

    rows = [r0, r1, r2, r3]

    def fire(j, k, gsem):
        pltpu.async_copy(x_hbm.at[src_v.at[j]], rows[k], gsem)

    def wait(k, gsem):
        pltpu.make_async_copy(x_hbm.at[src_v.at[k]], rows[k], gsem).wait()

    for k in range(4):
        fire(k, k, gsems[k])

    def quad(i, carry):
        j0 = 4 * i
        for k in range(4):
            wait(k, gsems[k])
            pltpu.sync_copy(rows[k], acc.at[dst_v.at[j0 + k]], add=True)

            @pl.when(j0 + k + 4 < CHUNKS_PER_S)
            def _():
                fire(j0 + k + 4, k, gsems[k])
        return carry

    lax.fori_loop(0, CHUNKS_PER_S // 4, quad, 0)
    plsc.subcore_barrier()
    pltpu.sync_copy(acc.at[pl.ds(row0, ROWS_PER_S)],
                    out_hbm.at[c, pl.ds(row0, ROWS_PER_S)])

    @pl.when(s == 0)
    def _():
        pltpu.sync_copy(acc.at[pl.ds(TAIL0, TAIL)],
                        out_hbm.at[c, pl.ds(TAIL0, TAIL)])


@functools.lru_cache(maxsize=1)
def _sc_agg():
    mesh = plsc.VectorSubcoreMesh(core_axis_name="c", subcore_axis_name="s")
    return pl.kernel(
        _sc_agg_body,
        out_type=jax.ShapeDtypeStruct((NC, N, DH), jnp.float32),
        mesh=mesh,
        scratch_types=[
            pltpu.VMEM((CHUNKS_PER_S, CHUNK), jnp.int32),
            pltpu.VMEM((CHUNKS_PER_S, CHUNK), jnp.int32),
            pltpu.VMEM((CHUNK, DH), jnp.float32),
            pltpu.VMEM((CHUNK, DH), jnp.float32),
            pltpu.VMEM((CHUNK, DH), jnp.float32),
            pltpu.VMEM((CHUNK, DH), jnp.float32),
            pltpu.VMEM_SHARED((N_PAD, DH), jnp.float32),
            pltpu.SemaphoreType.DMA,
            pltpu.SemaphoreType.DMA,
            pltpu.SemaphoreType.DMA,
            pltpu.SemaphoreType.DMA,
        ],
        compiler_params=pltpu.CompilerParams(use_tc_tiling_on_sc=False),
    )


def _tc_head_body(a_ref, b_ref, batch_ref,
                  w1_ref, b1_ref, w2_ref, b2_ref, bng_ref, bnb_ref,
                  fc1w_ref, fc1b_ref, fc2w_ref, fc2b_ref,
                  out_ref, acc_ref):
    i = pl.program_id(0)

    @pl.when(i == 0)
    def _():
        acc_ref[...] = jnp.zeros_like(acc_ref)

    h = jnp.concatenate([a_ref[...], b_ref[...]], axis=1)
    h = jnp.maximum(
        jax.lax.dot(h, w1_ref[...], preferred_element_type=jnp.float32)
        + b1_ref[...], 0.0)
    h = jax.lax.dot(h, w2_ref[...], preferred_element_type=jnp.float32) \
        + b2_ref[...]
    h = jnp.maximum(h, 0.0)
    h = h * (BN_INV * bng_ref[...]) + bnb_ref[...]

    bvec = batch_ref[0, 0, :]
    onehot = jnp.where(
        bvec[:, None] == jax.lax.broadcasted_iota(jnp.int32, (BLK, G), 1),
        1.0, 0.0)
    acc_ref[...] += jax.lax.dot_general(
        onehot, h, (((0,), (0,)), ((), ())),
        preferred_element_type=jnp.float32)

    @pl.when(i == NB - 1)
    def _():
        g = jnp.maximum(
            jax.lax.dot(acc_ref[...], fc1w_ref[...],
                        preferred_element_type=jnp.float32) + fc1b_ref[...],
            0.0)
        out_ref[...] = jax.lax.dot(
            g, fc2w_ref[...], preferred_element_type=jnp.float32) + fc2b_ref[...]


_tc_head = pl.pallas_call(
    _tc_head_body,
    grid=(NB,),
    in_specs=[
        pl.BlockSpec((BLK, DH), lambda i: (i, 0)),
        pl.BlockSpec((BLK, DH), lambda i: (i, 0)),
        pl.BlockSpec((1, 1, BLK), lambda i: (i, 0, 0)),
        pl.BlockSpec((D_IN, DIM), lambda i: (0, 0)),
        pl.BlockSpec((1, DIM), lambda i: (0, 0)),
        pl.BlockSpec((DIM, DIM), lambda i: (0, 0)),
        pl.BlockSpec((1, DIM), lambda i: (0, 0)),
        pl.BlockSpec((1, DIM), lambda i: (0, 0)),
        pl.BlockSpec((1, DIM), lambda i: (0, 0)),
        pl.BlockSpec((DIM, DIM), lambda i: (0, 0)),
        pl.BlockSpec((1, DIM), lambda i: (0, 0)),
        pl.BlockSpec((DIM, G), lambda i: (0, 0)),
        pl.BlockSpec((1, G), lambda i: (0, 0)),
    ],
    out_specs=pl.BlockSpec((G, G), lambda i: (0, 0)),
    out_shape=jax.ShapeDtypeStruct((G, G), jnp.float32),
    scratch_shapes=[pltpu.VMEM((G, DIM), jnp.float32)],
    compiler_params=pltpu.CompilerParams(
        dimension_semantics=("arbitrary",)),
)


def kernel(x, edge_index, batch, W1, b1, W2, b2, bn_g, bn_b,
           fc1_W, fc1_b, fc2_W, fc2_b):
    src = edge_index[0]
    dst = edge_index[1]
    pad = E_PAD - E
    src_p = jnp.concatenate([src, jnp.zeros((pad,), jnp.int32)])
    src_p = src_p.reshape(NS, CHUNKS_PER_S, CHUNK)
    src_p = jnp.concatenate([src_p, src_p + N])
    dst_p = jnp.concatenate(
        [dst, jnp.full((pad,), N, jnp.int32)]).reshape(NS, CHUNKS_PER_S, CHUNK)
    x2 = x.reshape(N, NC, DH).transpose(1, 0, 2).reshape(NC * N, DH)

    hio = _sc_agg()(x2, src_p, dst_p)

    batch3 = batch.reshape(NB, 1, BLK)

    fc2p = jnp.pad(fc2_W, ((0, 0), (0, G - 1)))
    fc2bp = jnp.pad(fc2_b, (0, G - 1)).reshape(1, G)

    out = _tc_head(hio[0], hio[1], batch3,
                   W1, b1.reshape(1, DIM), W2, b2.reshape(1, DIM),
                   bn_g.reshape(1, DIM), bn_b.reshape(1, DIM),
                   fc1_W, fc1_b.reshape(1, DIM), fc2p, fc2bp)
    return out[:, :1]


# speedup vs baseline: 2.5635x; 2.5262x over previous
import functools

import jax
import jax.numpy as jnp
from jax import lax
from jax.experimental import pallas as pl
from jax.experimental.pallas import tpu as pltpu
from jax.experimental.pallas import tpu_sc as plsc

N = 10000
E = 320000
D_IN = 128
DIM = 256
G = 128

NC = 2
NS = 16
DH = D_IN // NC
CHUNK = 128
CHUNKS_PER_S = 160
E_PAD = NS * CHUNKS_PER_S * CHUNK
N_DUMMY = 128
N_PAD = N + N_DUMMY
ROWS_PER_S = 624
TAIL0 = NS * ROWS_PER_S
TAIL = N - TAIL0

BLK = 400
NB = N // BLK
BN_INV = 1.0 / (1.0 + 1e-5) ** 0.5


def _sc_agg_body(x_hbm, src_hbm, dst_hbm, out_hbm,
                 src_v, dst_v, r0, r1, r2, r3, acc, *gsems):
    c = lax.axis_index("c")
    s = lax.axis_index("s")
    pltpu.sync_copy(src_hbm.at[c * NS + s], src_v)
    pltpu.sync_copy(dst_hbm.at[s], dst_v)
    row0 = s * ROWS_PER_S
    pltpu.sync_copy(x_hbm.at[pl.ds(c * N + row0, ROWS_PER_S)],
                    acc.at[pl.ds(row0, ROWS_PER_S)])

    @pl.when(s == 0)
    def _():
        pltpu.sync_copy(x_hbm.at[pl.ds(c * N + TAIL0, TAIL)],
                        acc.at[pl.ds(TAIL0, TAIL)])

    plsc.subcore_barrier()

    rows = [r0, r1, r2, r3]

    def fire(j, k, gsem):
        pltpu.async_copy(x_hbm.at[src_v.at[j]], rows[k], gsem)

    def wait(k, gsem):
        pltpu.make_async_copy(x_hbm.at[src_v.at[k]], rows[k], gsem).wait()

    for k in range(4):
        fire(k, k, gsems[k])

    def quad(i, carry):
        j0 = 4 * i
        for k in range(4):
            wait(k, gsems[k])
            pltpu.sync_copy(rows[k], acc.at[dst_v.at[j0 + k]], add=True)

            @pl.when(j0 + k + 4 < CHUNKS_PER_S)
            def _():
                fire(j0 + k + 4, k, gsems[k])
        return carry

    lax.fori_loop(0, CHUNKS_PER_S // 4, quad, 0)
    plsc.subcore_barrier()
    pltpu.sync_copy(acc.at[pl.ds(row0, ROWS_PER_S)],
                    out_hbm.at[c, pl.ds(row0, ROWS_PER_S)])

    @pl.when(s == 0)
    def _():
        pltpu.sync_copy(acc.at[pl.ds(TAIL0, TAIL)],
                        out_hbm.at[c, pl.ds(TAIL0, TAIL)])


@functools.lru_cache(maxsize=1)
def _sc_agg():
    mesh = plsc.VectorSubcoreMesh(core_axis_name="c", subcore_axis_name="s")
    return pl.kernel(
        _sc_agg_body,
        out_type=jax.ShapeDtypeStruct((NC, N, DH), jnp.float32),
        mesh=mesh,
        scratch_types=[
            pltpu.VMEM((CHUNKS_PER_S, CHUNK), jnp.int32),
            pltpu.VMEM((CHUNKS_PER_S, CHUNK), jnp.int32),
            pltpu.VMEM((CHUNK, DH), jnp.float32),
            pltpu.VMEM((CHUNK, DH), jnp.float32),
            pltpu.VMEM((CHUNK, DH), jnp.float32),
            pltpu.VMEM((CHUNK, DH), jnp.float32),
            pltpu.VMEM_SHARED((N_PAD, DH), jnp.float32),
            pltpu.SemaphoreType.DMA,
            pltpu.SemaphoreType.DMA,
            pltpu.SemaphoreType.DMA,
            pltpu.SemaphoreType.DMA,
        ],
        compiler_params=pltpu.CompilerParams(use_tc_tiling_on_sc=False),
    )


def _tc_head_body(a_ref, b_ref, batch_ref,
                  w1_ref, b1_ref, w2_ref, b2_ref, bng_ref, bnb_ref,
                  fc1w_ref, fc1b_ref, fc2w_ref, fc2b_ref,
                  out_ref, acc_ref):
    i = pl.program_id(0)

    @pl.when(i == 0)
    def _():
        acc_ref[...] = jnp.zeros_like(acc_ref)

    h = jnp.concatenate([a_ref[...], b_ref[...]], axis=1)
    h = jnp.maximum(
        jax.lax.dot(h, w1_ref[...], preferred_element_type=jnp.float32)
        + b1_ref[...], 0.0)
    h = jax.lax.dot(h, w2_ref[...], preferred_element_type=jnp.float32) \
        + b2_ref[...]
    h = jnp.maximum(h, 0.0)
    h = h * (BN_INV * bng_ref[...]) + bnb_ref[...]

    bvec = batch_ref[0, 0, :]
    onehot = jnp.where(
        bvec[:, None] == jax.lax.broadcasted_iota(jnp.int32, (BLK, G), 1),
        1.0, 0.0)
    acc_ref[...] += jax.lax.dot_general(
        onehot, h, (((0,), (0,)), ((), ())),
        preferred_element_type=jnp.float32)

    @pl.when(i == NB - 1)
    def _():
        g = jnp.maximum(
            jax.lax.dot(acc_ref[...], fc1w_ref[...],
                        preferred_element_type=jnp.float32) + fc1b_ref[...],
            0.0)
        out_ref[...] = jax.lax.dot(
            g, fc2w_ref[...], preferred_element_type=jnp.float32) + fc2b_ref[...]


_tc_head = pl.pallas_call(
    _tc_head_body,
    grid=(NB,),
    in_specs=[
        pl.BlockSpec((BLK, DH), lambda i: (i, 0)),
        pl.BlockSpec((BLK, DH), lambda i: (i, 0)),
        pl.BlockSpec((1, 1, BLK), lambda i: (i, 0, 0)),
        pl.BlockSpec((D_IN, DIM), lambda i: (0, 0)),
        pl.BlockSpec((1, DIM), lambda i: (0, 0)),
        pl.BlockSpec((DIM, DIM), lambda i: (0, 0)),
        pl.BlockSpec((1, DIM), lambda i: (0, 0)),
        pl.BlockSpec((1, DIM), lambda i: (0, 0)),
        pl.BlockSpec((1, DIM), lambda i: (0, 0)),
        pl.BlockSpec((DIM, DIM), lambda i: (0, 0)),
        pl.BlockSpec((1, DIM), lambda i: (0, 0)),
        pl.BlockSpec((DIM, G), lambda i: (0, 0)),
        pl.BlockSpec((1, G), lambda i: (0, 0)),
    ],
    out_specs=pl.BlockSpec((G, G), lambda i: (0, 0)),
    out_shape=jax.ShapeDtypeStruct((G, G), jnp.float32),
    scratch_shapes=[pltpu.VMEM((G, DIM), jnp.float32)],
    compiler_params=pltpu.CompilerParams(
        dimension_semantics=("arbitrary",)),
)


def kernel(x, edge_index, batch, W1, b1, W2, b2, bn_g, bn_b,
           fc1_W, fc1_b, fc2_W, fc2_b):
    src = edge_index[0]
    dst = edge_index[1]
    pad = E_PAD - E
    padv = jax.lax.iota(jnp.int32, pad)
    src_p = jnp.concatenate([src, padv % N])
    src_p = src_p.reshape(NS, CHUNKS_PER_S, CHUNK)
    src_p = jnp.concatenate([src_p, src_p + N])
    dst_p = jnp.concatenate(
        [dst, N + padv % N_DUMMY]).reshape(NS, CHUNKS_PER_S, CHUNK)
    x2 = x.reshape(N, NC, DH).transpose(1, 0, 2).reshape(NC * N, DH)

    hio = _sc_agg()(x2, src_p, dst_p)

    batch3 = batch.reshape(NB, 1, BLK)

    fc2p = jnp.pad(fc2_W, ((0, 0), (0, G - 1)))
    fc2bp = jnp.pad(fc2_b, (0, G - 1)).reshape(1, G)

    out = _tc_head(hio[0], hio[1], batch3,
                   W1, b1.reshape(1, DIM), W2, b2.reshape(1, DIM),
                   bn_g.reshape(1, DIM), bn_b.reshape(1, DIM),
                   fc1_W, fc1_b.reshape(1, DIM), fc2p, fc2bp)
    return out[:, :1]


# speedup vs baseline: 2.5659x; 1.0009x over previous
import functools

import jax
import jax.numpy as jnp
from jax import lax
from jax.experimental import pallas as pl
from jax.experimental.pallas import tpu as pltpu
from jax.experimental.pallas import tpu_sc as plsc

N = 10000
E = 320000
D_IN = 128
DIM = 256
G = 128

NC = 2
NS = 16
DH = D_IN // NC
CHUNK = 128
CHUNKS_PER_S = 160
E_PAD = NS * CHUNKS_PER_S * CHUNK
N_DUMMY = 128
N_PAD = N + N_DUMMY
ROWS_PER_S = 624
TAIL0 = NS * ROWS_PER_S
TAIL = N - TAIL0

BLK = 400
NB = N // BLK
BN_INV = 1.0 / (1.0 + 1e-5) ** 0.5


def _sc_agg_body(x_hbm, src_hbm, dst_hbm, out_hbm,
                 src_v, dst_v, r0, r1, r2, r3, acc, *gsems):
    c = lax.axis_index("c")
    s = lax.axis_index("s")
    pltpu.sync_copy(src_hbm.at[c * NS + s], src_v)
    pltpu.sync_copy(dst_hbm.at[s], dst_v)
    row0 = s * ROWS_PER_S
    pltpu.sync_copy(x_hbm.at[pl.ds(c * N + row0, ROWS_PER_S)],
                    acc.at[pl.ds(row0, ROWS_PER_S)])

    @pl.when(s == 0)
    def _():
        pltpu.sync_copy(x_hbm.at[pl.ds(c * N + TAIL0, TAIL)],
                        acc.at[pl.ds(TAIL0, TAIL)])

    plsc.subcore_barrier()

    rows = [r0, r1, r2, r3]

    def fire(j, k, gsem):
        pltpu.async_copy(x_hbm.at[src_v.at[j]], rows[k], gsem)

    def wait(k, gsem):
        pltpu.make_async_copy(x_hbm.at[src_v.at[k]], rows[k], gsem).wait()

    for k in range(4):
        fire(k, k, gsems[k])

    def quad(i, carry):
        j0 = 4 * i
        for k in range(4):
            wait(k, gsems[k])
            pltpu.sync_copy(rows[k], acc.at[dst_v.at[j0 + k]], add=True)

            @pl.when(j0 + k + 4 < CHUNKS_PER_S)
            def _():
                fire(j0 + k + 4, k, gsems[k])
        return carry

    lax.fori_loop(0, CHUNKS_PER_S // 4, quad, 0)
    plsc.subcore_barrier()
    pltpu.sync_copy(acc.at[pl.ds(row0, ROWS_PER_S)],
                    out_hbm.at[c, pl.ds(row0, ROWS_PER_S)])

    @pl.when(s == 0)
    def _():
        pltpu.sync_copy(acc.at[pl.ds(TAIL0, TAIL)],
                        out_hbm.at[c, pl.ds(TAIL0, TAIL)])


@functools.lru_cache(maxsize=1)
def _sc_agg():
    mesh = plsc.VectorSubcoreMesh(core_axis_name="c", subcore_axis_name="s")
    return pl.kernel(
        _sc_agg_body,
        out_type=jax.ShapeDtypeStruct((NC, N, DH), jnp.float32),
        mesh=mesh,
        scratch_types=[
            pltpu.VMEM((CHUNKS_PER_S, CHUNK), jnp.int32),
            pltpu.VMEM((CHUNKS_PER_S, CHUNK), jnp.int32),
            pltpu.VMEM((CHUNK, DH), jnp.float32),
            pltpu.VMEM((CHUNK, DH), jnp.float32),
            pltpu.VMEM((CHUNK, DH), jnp.float32),
            pltpu.VMEM((CHUNK, DH), jnp.float32),
            pltpu.VMEM_SHARED((N_PAD, DH), jnp.float32),
            pltpu.SemaphoreType.DMA,
            pltpu.SemaphoreType.DMA,
            pltpu.SemaphoreType.DMA,
            pltpu.SemaphoreType.DMA,
        ],
        compiler_params=pltpu.CompilerParams(use_tc_tiling_on_sc=False),
    )


def _tc_head_body(a_ref, b_ref, batch_ref,
                  w1_ref, b1_ref, w2_ref, b2_ref, bng_ref, bnb_ref,
                  fc1w_ref, fc1b_ref, fc2w_ref, fc2b_ref,
                  out_ref, acc_ref):
    i = pl.program_id(0)

    @pl.when(i == 0)
    def _():
        acc_ref[...] = jnp.zeros_like(acc_ref)

    h = jnp.concatenate([a_ref[...], b_ref[...]], axis=1)
    h = jnp.maximum(
        jax.lax.dot(h.astype(jnp.bfloat16), w1_ref[...],
                    preferred_element_type=jnp.float32)
        + b1_ref[...], 0.0)
    h = jax.lax.dot(h.astype(jnp.bfloat16), w2_ref[...],
                    preferred_element_type=jnp.float32) \
        + b2_ref[...]
    h = jnp.maximum(h, 0.0)
    h = h * (BN_INV * bng_ref[...]) + bnb_ref[...]

    bvec = batch_ref[0, 0, :]
    onehot = jnp.where(
        bvec[:, None] == jax.lax.broadcasted_iota(jnp.int32, (BLK, G), 1),
        1.0, 0.0)
    acc_ref[...] += jax.lax.dot_general(
        onehot, h, (((0,), (0,)), ((), ())),
        preferred_element_type=jnp.float32)

    @pl.when(i == NB - 1)
    def _():
        g = jnp.maximum(
            jax.lax.dot(acc_ref[...], fc1w_ref[...],
                        preferred_element_type=jnp.float32) + fc1b_ref[...],
            0.0)
        out_ref[...] = jax.lax.dot(
            g, fc2w_ref[...], preferred_element_type=jnp.float32) + fc2b_ref[...]


_tc_head = pl.pallas_call(
    _tc_head_body,
    grid=(NB,),
    in_specs=[
        pl.BlockSpec((BLK, DH), lambda i: (i, 0)),
        pl.BlockSpec((BLK, DH), lambda i: (i, 0)),
        pl.BlockSpec((1, 1, BLK), lambda i: (i, 0, 0)),
        pl.BlockSpec((D_IN, DIM), lambda i: (0, 0)),
        pl.BlockSpec((1, DIM), lambda i: (0, 0)),
        pl.BlockSpec((DIM, DIM), lambda i: (0, 0)),
        pl.BlockSpec((1, DIM), lambda i: (0, 0)),
        pl.BlockSpec((1, DIM), lambda i: (0, 0)),
        pl.BlockSpec((1, DIM), lambda i: (0, 0)),
        pl.BlockSpec((DIM, DIM), lambda i: (0, 0)),
        pl.BlockSpec((1, DIM), lambda i: (0, 0)),
        pl.BlockSpec((DIM, G), lambda i: (0, 0)),
        pl.BlockSpec((1, G), lambda i: (0, 0)),
    ],
    out_specs=pl.BlockSpec((G, G), lambda i: (0, 0)),
    out_shape=jax.ShapeDtypeStruct((G, G), jnp.float32),
    scratch_shapes=[pltpu.VMEM((G, DIM), jnp.float32)],
    compiler_params=pltpu.CompilerParams(
        dimension_semantics=("arbitrary",)),
)


def kernel(x, edge_index, batch, W1, b1, W2, b2, bn_g, bn_b,
           fc1_W, fc1_b, fc2_W, fc2_b):
    src = edge_index[0]
    dst = edge_index[1]
    pad = E_PAD - E
    padv = jax.lax.iota(jnp.int32, pad)
    src_p = jnp.concatenate([src, padv % N])
    src_p = src_p.reshape(NS, CHUNKS_PER_S, CHUNK)
    src_p = jnp.concatenate([src_p, src_p + N])
    dst_p = jnp.concatenate(
        [dst, N + padv % N_DUMMY]).reshape(NS, CHUNKS_PER_S, CHUNK)
    x2 = x.reshape(N, NC, DH).transpose(1, 0, 2).reshape(NC * N, DH)

    hio = _sc_agg()(x2, src_p, dst_p)

    batch3 = batch.reshape(NB, 1, BLK)

    fc2p = jnp.pad(fc2_W, ((0, 0), (0, G - 1)))
    fc2bp = jnp.pad(fc2_b, (0, G - 1)).reshape(1, G)

    out = _tc_head(hio[0], hio[1], batch3,
                   W1.astype(jnp.bfloat16), b1.reshape(1, DIM),
                   W2.astype(jnp.bfloat16), b2.reshape(1, DIM),
                   bn_g.reshape(1, DIM), bn_b.reshape(1, DIM),
                   fc1_W, fc1_b.reshape(1, DIM), fc2p, fc2bp)
    return out[:, :1]


# speedup vs baseline: 3.0122x; 1.1739x over previous
import functools

import jax
import jax.numpy as jnp
from jax import lax
from jax.experimental import pallas as pl
from jax.experimental.pallas import tpu as pltpu
from jax.experimental.pallas import tpu_sc as plsc

N = 10000
E = 320000
D_IN = 128
DIM = 256
G = 128

NC = 2
NS = 16
DH = D_IN // NC
CHUNK = 128
CHUNKS_PER_S = 160
E_PAD = NS * CHUNKS_PER_S * CHUNK
N_DUMMY = 128
N_PAD = N + N_DUMMY
ROWS_PER_S = 624
TAIL0 = NS * ROWS_PER_S
TAIL = N - TAIL0

BLK = 400
NB = N // BLK
BN_INV = 1.0 / (1.0 + 1e-5) ** 0.5


def _sc_agg_body(x_hbm, ei_hbm, out_hbm,
                 src_v, dst_v, idx4, r0, r1, r2, r3, acc, *gsems):
    c = lax.axis_index("c")
    s = lax.axis_index("s")
    pltpu.sync_copy(ei_hbm.at[0, s], src_v)
    pltpu.sync_copy(ei_hbm.at[1, s], dst_v)
    row0 = s * ROWS_PER_S

    def zrow(i, carry):
        for q in range(DH // 16):
            r0[i, pl.ds(16 * q, 16)] = jnp.zeros((16,), jnp.float32)
        return carry

    lax.fori_loop(0, CHUNK, zrow, 0)
    for t in range(ROWS_PER_S // CHUNK):
        pltpu.sync_copy(r0, acc.at[pl.ds(row0 + t * CHUNK, CHUNK)])
    rem = ROWS_PER_S % CHUNK
    pltpu.sync_copy(r0.at[pl.ds(0, rem)],
                    acc.at[pl.ds(row0 + ROWS_PER_S - rem, rem)])

    @pl.when(s == 0)
    def _():
        pltpu.sync_copy(r0.at[pl.ds(0, TAIL)], acc.at[pl.ds(TAIL0, TAIL)])

    plsc.subcore_barrier()

    rows = [r0, r1, r2, r3]

    def fire(j, k, gsem):
        for q in range(CHUNK // 16):
            idx4[k, pl.ds(16 * q, 16)] = src_v[j, pl.ds(16 * q, 16)] * 2 + c
        pltpu.async_copy(x_hbm.at[idx4.at[k]], rows[k], gsem)

    def wait(k, gsem):
        pltpu.make_async_copy(x_hbm.at[idx4.at[k]], rows[k], gsem).wait()

    for k in range(4):
        fire(k, k, gsems[k])

    def quad(i, carry):
        j0 = 4 * i
        for k in range(4):
            wait(k, gsems[k])
            pltpu.sync_copy(rows[k], acc.at[dst_v.at[j0 + k]], add=True)

            @pl.when(j0 + k + 4 < CHUNKS_PER_S)
            def _():
                fire(j0 + k + 4, k, gsems[k])
        return carry

    lax.fori_loop(0, CHUNKS_PER_S // 4, quad, 0)
    plsc.subcore_barrier()
    pltpu.sync_copy(acc.at[pl.ds(row0, ROWS_PER_S)],
                    out_hbm.at[c, pl.ds(row0, ROWS_PER_S)])

    @pl.when(s == 0)
    def _():
        pltpu.sync_copy(acc.at[pl.ds(TAIL0, TAIL)],
                        out_hbm.at[c, pl.ds(TAIL0, TAIL)])


@functools.lru_cache(maxsize=1)
def _sc_agg():
    mesh = plsc.VectorSubcoreMesh(core_axis_name="c", subcore_axis_name="s")
    return pl.kernel(
        _sc_agg_body,
        out_type=jax.ShapeDtypeStruct((NC, N, DH), jnp.float32),
        mesh=mesh,
        scratch_types=[
            pltpu.VMEM((CHUNKS_PER_S, CHUNK), jnp.int32),
            pltpu.VMEM((CHUNKS_PER_S, CHUNK), jnp.int32),
            pltpu.VMEM((4, CHUNK), jnp.int32),
            pltpu.VMEM((CHUNK, DH), jnp.float32),
            pltpu.VMEM((CHUNK, DH), jnp.float32),
            pltpu.VMEM((CHUNK, DH), jnp.float32),
            pltpu.VMEM((CHUNK, DH), jnp.float32),
            pltpu.VMEM_SHARED((N_PAD, DH), jnp.float32),
            pltpu.SemaphoreType.DMA,
            pltpu.SemaphoreType.DMA,
            pltpu.SemaphoreType.DMA,
            pltpu.SemaphoreType.DMA,
        ],
        compiler_params=pltpu.CompilerParams(use_tc_tiling_on_sc=False),
    )


def _tc_head_body(x_ref, a_ref, b_ref, batch_ref,
                  w1_ref, b1_ref, w2_ref, b2_ref, bng_ref, bnb_ref,
                  fc1w_ref, fc1b_ref, fc2w_ref, fc2b_ref,
                  out_ref, acc_ref):
    i = pl.program_id(0)

    @pl.when(i == 0)
    def _():
        acc_ref[...] = jnp.zeros_like(acc_ref)

    h = x_ref[...] + jnp.concatenate([a_ref[...], b_ref[...]], axis=1)
    h = jnp.maximum(
        jax.lax.dot(h.astype(jnp.bfloat16), w1_ref[...],
                    preferred_element_type=jnp.float32)
        + b1_ref[...], 0.0)
    h = jax.lax.dot(h.astype(jnp.bfloat16), w2_ref[...],
                    preferred_element_type=jnp.float32) \
        + b2_ref[...]
    h = jnp.maximum(h, 0.0)
    h = h * (BN_INV * bng_ref[...]) + bnb_ref[...]

    bvec = batch_ref[0, 0, :]
    onehot = jnp.where(
        bvec[:, None] == jax.lax.broadcasted_iota(jnp.int32, (BLK, G), 1),
        1.0, 0.0)
    acc_ref[...] += jax.lax.dot_general(
        onehot, h, (((0,), (0,)), ((), ())),
        preferred_element_type=jnp.float32)

    @pl.when(i == NB - 1)
    def _():
        g = jnp.maximum(
            jax.lax.dot(acc_ref[...], fc1w_ref[...],
                        preferred_element_type=jnp.float32) + fc1b_ref[...],
            0.0)
        out_ref[...] = jax.lax.dot(
            g, fc2w_ref[...], preferred_element_type=jnp.float32) + fc2b_ref[...]


_tc_head = pl.pallas_call(
    _tc_head_body,
    grid=(NB,),
    in_specs=[
        pl.BlockSpec((BLK, D_IN), lambda i: (i, 0)),
        pl.BlockSpec((BLK, DH), lambda i: (i, 0)),
        pl.BlockSpec((BLK, DH), lambda i: (i, 0)),
        pl.BlockSpec((1, 1, BLK), lambda i: (i, 0, 0)),
        pl.BlockSpec((D_IN, DIM), lambda i: (0, 0)),
        pl.BlockSpec((1, DIM), lambda i: (0, 0)),
        pl.BlockSpec((DIM, DIM), lambda i: (0, 0)),
        pl.BlockSpec((1, DIM), lambda i: (0, 0)),
        pl.BlockSpec((1, DIM), lambda i: (0, 0)),
        pl.BlockSpec((1, DIM), lambda i: (0, 0)),
        pl.BlockSpec((DIM, DIM), lambda i: (0, 0)),
        pl.BlockSpec((1, DIM), lambda i: (0, 0)),
        pl.BlockSpec((DIM, G), lambda i: (0, 0)),
        pl.BlockSpec((1, G), lambda i: (0, 0)),
    ],
    out_specs=pl.BlockSpec((G, G), lambda i: (0, 0)),
    out_shape=jax.ShapeDtypeStruct((G, G), jnp.float32),
    scratch_shapes=[pltpu.VMEM((G, DIM), jnp.float32)],
    compiler_params=pltpu.CompilerParams(
        dimension_semantics=("arbitrary",)),
)


def kernel(x, edge_index, batch, W1, b1, W2, b2, bn_g, bn_b,
           fc1_W, fc1_b, fc2_W, fc2_b):
    pad = E_PAD - E
    padv = jax.lax.iota(jnp.int32, pad)
    padrows = jnp.stack([padv % N, N + padv % N_DUMMY])
    ei_p = jnp.concatenate([edge_index, padrows], axis=1)
    ei_p = ei_p.reshape(2, NS, CHUNKS_PER_S, CHUNK)
    x2 = x.reshape(NC * N, DH)

    hio = _sc_agg()(x2, ei_p)

    batch3 = batch.reshape(NB, 1, BLK)

    fc2p = jnp.pad(fc2_W, ((0, 0), (0, G - 1)))
    fc2bp = jnp.pad(fc2_b, (0, G - 1)).reshape(1, G)

    out = _tc_head(x, hio[0], hio[1], batch3,
                   W1.astype(jnp.bfloat16), b1.reshape(1, DIM),
                   W2.astype(jnp.bfloat16), b2.reshape(1, DIM),
                   bn_g.reshape(1, DIM), bn_b.reshape(1, DIM),
                   fc1_W, fc1_b.reshape(1, DIM), fc2p, fc2bp)
    return out[:, :1]
